# Initial kernel scaffold; baseline (speedup 1.0000x reference)
import jax
import jax.numpy as jnp
from jax.experimental import pallas as pl


def kernel(x, edge_index, W_rel, b_rel, W_root):
    raise NotImplementedError("write your pallas kernel here")


# speedup vs baseline: 6.7316x; 6.7316x over previous
import functools

import jax
import jax.numpy as jnp
from jax import lax
from jax.experimental import pallas as pl
from jax.experimental.pallas import tpu as pltpu
from jax.experimental.pallas import tpu_sc as plsc

_NC = 2
_NS = 16
_NW = _NC * _NS

_CH = 128


def _sc_aggregate(x, src, dst, zeros):
    n, d = x.shape
    e = src.shape[0]
    nchunk = e // _CH
    maxit = (nchunk + _NW - 1) // _NW
    rows_per_tile = n // _NS

    mesh = plsc.VectorSubcoreMesh(core_axis_name="c", subcore_axis_name="s")

    @functools.partial(
        pl.kernel,
        out_type=jax.ShapeDtypeStruct((_NC, n, d), jnp.float32),
        mesh=mesh,
        scratch_types=[
            pltpu.VMEM((_CH,), jnp.int32),
            pltpu.VMEM((_CH,), jnp.int32),
            pltpu.VMEM((_CH, d), jnp.float32),
            pltpu.VMEM_SHARED((n, d), jnp.float32),
            pltpu.SemaphoreType.DMA,
        ],
        compiler_params=pltpu.CompilerParams(use_tc_tiling_on_sc=False),
    )
    def agg(x_hbm, src_hbm, dst_hbm, z_hbm, out_hbm,
            src_v, dst_v, rows_v, acc_sh, sem):
        c = lax.axis_index("c")
        s = lax.axis_index("s")
        wid = s * _NC + c
        pltpu.sync_copy(z_hbm, acc_sh.at[pl.ds(s * rows_per_tile, rows_per_tile)])
        plsc.subcore_barrier()

        def body(k, carry):
            chunk = k * _NW + wid

            @pl.when(chunk < nchunk)
            def _():
                base = chunk * _CH
                pltpu.sync_copy(src_hbm.at[pl.ds(base, _CH)], src_v)
                pltpu.sync_copy(dst_hbm.at[pl.ds(base, _CH)], dst_v)
                pltpu.async_copy(x_hbm.at[src_v], rows_v, sem).wait()
                pltpu.sync_copy(rows_v, acc_sh.at[dst_v], add=True)

            return carry

        lax.fori_loop(0, maxit, body, 0)
        plsc.subcore_barrier()
        pltpu.sync_copy(
            acc_sh.at[pl.ds(s * rows_per_tile, rows_per_tile)],
            out_hbm.at[c, pl.ds(s * rows_per_tile, rows_per_tile)],
        )

    return agg(x, src, dst, zeros)


def _tc_tail(partials, x, w_rel, w_root, b_rel):
    n, d = x.shape
    bn = 2000
    grid = (n // bn,)

    def body(p_ref, x_ref, wr_ref, wt_ref, b_ref, o_ref):
        aggr = p_ref[0] + p_ref[1]
        acc = lax.dot_general(aggr, wr_ref[...], (((1,), (1,)), ((), ())),
                              preferred_element_type=jnp.float32)
        acc += lax.dot_general(x_ref[...], wt_ref[...], (((1,), (1,)), ((), ())),
                               preferred_element_type=jnp.float32)
        o_ref[...] = jnp.maximum(acc + b_ref[...], 0.0)

    return pl.pallas_call(
        body,
        grid=grid,
        in_specs=[
            pl.BlockSpec((_NC, bn, d), lambda i: (0, i, 0)),
            pl.BlockSpec((bn, d), lambda i: (i, 0)),
            pl.BlockSpec((d, d), lambda i: (0, 0)),
            pl.BlockSpec((d, d), lambda i: (0, 0)),
            pl.BlockSpec((1, d), lambda i: (0, 0)),
        ],
        out_specs=pl.BlockSpec((bn, d), lambda i: (i, 0)),
        out_shape=jax.ShapeDtypeStruct((n, d), jnp.float32),
    )(partials, x, w_rel, w_root, b_rel)


def kernel(x, edge_index, W_rel, b_rel, W_root):
    n, d = x.shape
    src = edge_index[0]
    dst = edge_index[1]
    zeros = jnp.zeros((n // _NS, d), jnp.float32)
    partials = _sc_aggregate(x, src, dst, zeros)
    return _tc_tail(partials, x, W_rel, W_root, b_rel.reshape(1, d))


# speedup vs baseline: 10.3061x; 1.5310x over previous
import functools

import jax
import jax.numpy as jnp
from jax import lax
from jax.experimental import pallas as pl
from jax.experimental.pallas import tpu as pltpu
from jax.experimental.pallas import tpu_sc as plsc

_NC = 2
_NS = 16

_CH = 128


def _sc_aggregate(x0, x1, src2, dst2, zeros, nchunk):
    n, dh = x0.shape
    base_chunks = nchunk // _NS
    extra = nchunk - base_chunks * _NS
    maxrows = base_chunks + (1 if extra else 0)
    npairs = (maxrows + 1) // 2
    rows_per_tile = n // _NS

    mesh = plsc.VectorSubcoreMesh(core_axis_name="c", subcore_axis_name="s")

    @functools.partial(
        pl.kernel,
        out_type=jax.ShapeDtypeStruct((_NC, n, dh), jnp.float32),
        mesh=mesh,
        scratch_types=[
            pltpu.VMEM((maxrows, _CH), jnp.int32),
            pltpu.VMEM((maxrows, _CH), jnp.int32),
            pltpu.VMEM((2, _CH, dh), jnp.float32),
            pltpu.VMEM_SHARED((n, dh), jnp.float32),
            pltpu.SemaphoreType.DMA,
        ],
        compiler_params=pltpu.CompilerParams(use_tc_tiling_on_sc=False),
    )
    def agg(x0_hbm, x1_hbm, src_hbm, dst_hbm, z_hbm, out_hbm,
            sidx, didx, rows, acc_sh, gsem):
        c = lax.axis_index("c")
        s = lax.axis_index("s")
        base = s * base_chunks + jnp.minimum(s, extra)
        cnt = base_chunks + (s < extra).astype(jnp.int32)
        pltpu.sync_copy(z_hbm, acc_sh.at[pl.ds(s * rows_per_tile, rows_per_tile)])
        pltpu.sync_copy(src_hbm.at[pl.ds(base, maxrows)], sidx)
        pltpu.sync_copy(dst_hbm.at[pl.ds(base, maxrows)], didx)
        plsc.subcore_barrier()

        def gather(k, buf):
            @pl.when(c == 0)
            def _():
                pltpu.async_copy(x0_hbm.at[sidx.at[k]], rows.at[buf], gsem)

            @pl.when(c == 1)
            def _():
                pltpu.async_copy(x1_hbm.at[sidx.at[k]], rows.at[buf], gsem)

        def gather_wait(k, buf):
            pltpu.make_async_copy(x0_hbm.at[sidx.at[k]], rows.at[buf], gsem).wait()

        gather(0, 0)

        def pair(k2, carry):
            for b in range(2):
                k = k2 * 2 + b
                nxt = k + 1

                @pl.when(nxt < cnt)
                def _():
                    gather(nxt, 1 - b)

                @pl.when(k < cnt)
                def _():
                    gather_wait(k, b)
                    pltpu.sync_copy(rows.at[b], acc_sh.at[didx.at[k]], add=True)

            return carry

        lax.fori_loop(0, npairs, pair, 0)
        plsc.subcore_barrier()
        pltpu.sync_copy(
            acc_sh.at[pl.ds(s * rows_per_tile, rows_per_tile)],
            out_hbm.at[c, pl.ds(s * rows_per_tile, rows_per_tile)],
        )

    return agg(x0, x1, src2, dst2, zeros)


def _tc_tail(partials, x, w_rel, w_root, b_rel):
    n, d = x.shape
    dh = d // 2
    bn = 2000
    grid = (n // bn,)

    def body(p_ref, x_ref, wr_ref, wt_ref, b_ref, o_ref):
        aggr = jnp.concatenate([p_ref[0], p_ref[1]], axis=-1)
        acc = lax.dot_general(aggr, wr_ref[...], (((1,), (1,)), ((), ())),
                              preferred_element_type=jnp.float32)
        acc += lax.dot_general(x_ref[...], wt_ref[...], (((1,), (1,)), ((), ())),
                               preferred_element_type=jnp.float32)
        o_ref[...] = jnp.maximum(acc + b_ref[...], 0.0)

    return pl.pallas_call(
        body,
        grid=grid,
        in_specs=[
            pl.BlockSpec((_NC, bn, dh), lambda i: (0, i, 0)),
            pl.BlockSpec((bn, d), lambda i: (i, 0)),
            pl.BlockSpec((d, d), lambda i: (0, 0)),
            pl.BlockSpec((d, d), lambda i: (0, 0)),
            pl.BlockSpec((1, d), lambda i: (0, 0)),
        ],
        out_specs=pl.BlockSpec((bn, d), lambda i: (i, 0)),
        out_shape=jax.ShapeDtypeStruct((n, d), jnp.float32),
    )(partials, x, w_rel, w_root, b_rel)


def kernel(x, edge_index, W_rel, b_rel, W_root):
    n, d = x.shape
    dh = d // 2
    e = edge_index.shape[1]
    nchunk = e // _CH
    nchunk_pad = nchunk + 8
    pad = nchunk_pad * _CH - e
    ei = jnp.concatenate(
        [edge_index, jnp.zeros((2, pad), jnp.int32)], axis=1
    ).reshape(2, nchunk_pad, _CH)
    x0 = x[:, :dh]
    x1 = x[:, dh:]
    zeros = jnp.zeros((n // _NS, dh), jnp.float32)
    partials = _sc_aggregate(x0, x1, ei[0], ei[1], zeros, nchunk)
    return _tc_tail(partials, x, W_rel, W_root, b_rel.reshape(1, d))
